# baseline (speedup 1.0000x reference)
import jax
import jax.numpy as jnp
from jax.experimental import pallas as pl
from jax.experimental.pallas import tpu as pltpu

_N = 10000
_NH = 128

_IB = 200
_NI = _N // _IB
_NBUF = 4


def _agg_body(adj_ref, x_ref, w_ref, b_ref, o_ref,
              buf_ref, xv_ref, acc_ref, deg_ref, sem, xsem):
    def start_copy(k, slot):
        pltpu.make_async_copy(
            adj_ref.at[pl.ds(k * _IB, _IB), :],
            buf_ref.at[slot],
            sem.at[slot],
        ).start()

    xcopy = pltpu.make_async_copy(x_ref, xv_ref, xsem)
    xcopy.start()
    for k in range(_NBUF):
        start_copy(k, k)
    xcopy.wait()

    for k in range(_NI):
        slot = k % _NBUF
        pltpu.make_async_copy(
            adj_ref.at[pl.ds(k * _IB, _IB), :],
            buf_ref.at[slot],
            sem.at[slot],
        ).wait()
        sup = jnp.maximum(
            jax.lax.dot_general(
                xv_ref[k * _IB:(k + 1) * _IB, :], w_ref[...],
                (((1,), (1,)), ((), ())),
                preferred_element_type=jnp.float32) + b_ref[...],
            0.0).astype(jnp.bfloat16)
        sel = jnp.where(buf_ref[slot] > 0.0, 1.0, 0.0)
        dsum = jnp.sum(sel, axis=0, keepdims=True)
        mask = sel.astype(jnp.bfloat16)
        part = jax.lax.dot_general(
            sup, mask, (((0,), (0,)), ((), ())),
            preferred_element_type=jnp.float32)
        if k == 0:
            deg_ref[...] = dsum
            acc_ref[...] = part
        else:
            deg_ref[...] += dsum
            acc_ref[...] += part
        if k + _NBUF < _NI:
            start_copy(k + _NBUF, slot)

    o_ref[...] = jnp.transpose(acc_ref[...] / deg_ref[...])


def kernel(input, adj, W, b):
    return pl.pallas_call(
        _agg_body,
        in_specs=[
            pl.BlockSpec(memory_space=pl.ANY),
            pl.BlockSpec(memory_space=pl.ANY),
            pl.BlockSpec(memory_space=pltpu.MemorySpace.VMEM),
            pl.BlockSpec(memory_space=pltpu.MemorySpace.VMEM),
        ],
        out_specs=pl.BlockSpec(memory_space=pltpu.MemorySpace.VMEM),
        out_shape=jax.ShapeDtypeStruct((_N, _NH), jnp.float32),
        scratch_shapes=[
            pltpu.VMEM((_NBUF, _IB, _N), jnp.float32),
            pltpu.VMEM((_N, _NH), jnp.float32),
            pltpu.VMEM((_NH, _N), jnp.float32),
            pltpu.VMEM((1, _N), jnp.float32),
            pltpu.SemaphoreType.DMA((_NBUF,)),
            pltpu.SemaphoreType.DMA,
        ],
    )(adj, input, W, b.reshape(1, _NH))
